# baseline (speedup 1.0000x reference)
import functools

import jax
import jax.numpy as jnp
from jax import lax
from jax.experimental import pallas as pl
from jax.experimental.pallas import tpu as pltpu

N_NODES = 10000
N_EDGES = 160000
NUM_REL = 2
NUM_GRAPHS = 64
HIDDEN = 512
NP = 10240
RB = 320


def _elu(x):
    return jnp.where(x > 0, x, jnp.exp(jnp.minimum(x, 0.0)) - 1.0)


def _mid_layer_body(h_ref, agg_ref, inv_ref, root_ref, W_ref, b_ref, out_ref):
    h = h_ref[...]
    out = jnp.dot(h, root_ref[...], preferred_element_type=jnp.float32)
    for r in range(NUM_REL):
        a = agg_ref[:, r, :] * inv_ref[:, r][:, None]
        out = out + jnp.dot(a, W_ref[r], preferred_element_type=jnp.float32)
    out = out + b_ref[...]
    out_ref[...] = _elu(out)


def _tc_mid_layer(h, agg, invcnt, root, W, b):
    din = h.shape[1]
    return pl.pallas_call(
        _mid_layer_body,
        grid=(NP // RB,),
        in_specs=[
            pl.BlockSpec((RB, din), lambda i: (i, 0)),
            pl.BlockSpec((RB, NUM_REL, din), lambda i: (i, 0, 0)),
            pl.BlockSpec((RB, NUM_REL), lambda i: (i, 0)),
            pl.BlockSpec((din, HIDDEN), lambda i: (0, 0)),
            pl.BlockSpec((NUM_REL, din, HIDDEN), lambda i: (0, 0, 0)),
            pl.BlockSpec((1, HIDDEN), lambda i: (0, 0)),
        ],
        out_specs=pl.BlockSpec((RB, HIDDEN), lambda i: (i, 0)),
        out_shape=jax.ShapeDtypeStruct((NP, HIDDEN), jnp.float32),
    )(h, agg, invcnt, root, W, b)


def _last_layer_body(h_ref, agg_ref, inv_ref, root_ref, W_ref, b_ref,
                     oh_ref, lw_ref, lb_ref, out_ref, sum_scr, cnt_scr):
    i = pl.program_id(0)

    @pl.when(i == 0)
    def _():
        sum_scr[...] = jnp.zeros_like(sum_scr)
        cnt_scr[...] = jnp.zeros_like(cnt_scr)

    h = h_ref[...]
    out = jnp.dot(h, root_ref[...], preferred_element_type=jnp.float32)
    for r in range(NUM_REL):
        a = agg_ref[:, r, :] * inv_ref[:, r][:, None]
        out = out + jnp.dot(a, W_ref[r], preferred_element_type=jnp.float32)
    hout = _elu(out + b_ref[...])

    oh = oh_ref[...]
    dn = (((0,), (0,)), ((), ()))
    sum_scr[...] += lax.dot_general(oh, hout, dn,
                                    preferred_element_type=jnp.float32)
    cnt_scr[...] += lax.dot_general(
        oh, jnp.ones((RB, 128), jnp.float32), dn,
        preferred_element_type=jnp.float32)

    @pl.when(i == pl.num_programs(0) - 1)
    def _():
        pooled = sum_scr[...] / jnp.maximum(cnt_scr[:, :1], 1.0)
        out_ref[...] = (jnp.dot(pooled, lw_ref[...],
                                preferred_element_type=jnp.float32)
                        + lb_ref[...])


def _tc_last_layer(h, agg, invcnt, root, W, b, onehotT, lin_w, lin_b):
    din = h.shape[1]
    ncls = lin_w.shape[1]
    return pl.pallas_call(
        _last_layer_body,
        grid=(NP // RB,),
        in_specs=[
            pl.BlockSpec((RB, din), lambda i: (i, 0)),
            pl.BlockSpec((RB, NUM_REL, din), lambda i: (i, 0, 0)),
            pl.BlockSpec((RB, NUM_REL), lambda i: (i, 0)),
            pl.BlockSpec((din, HIDDEN), lambda i: (0, 0)),
            pl.BlockSpec((NUM_REL, din, HIDDEN), lambda i: (0, 0, 0)),
            pl.BlockSpec((1, HIDDEN), lambda i: (0, 0)),
            pl.BlockSpec((RB, NUM_GRAPHS), lambda i: (i, 0)),
            pl.BlockSpec((HIDDEN, ncls), lambda i: (0, 0)),
            pl.BlockSpec((1, ncls), lambda i: (0, 0)),
        ],
        out_specs=pl.BlockSpec((NUM_GRAPHS, ncls), lambda i: (0, 0)),
        out_shape=jax.ShapeDtypeStruct((NUM_GRAPHS, ncls), jnp.float32),
        scratch_shapes=[
            pltpu.VMEM((NUM_GRAPHS, HIDDEN), jnp.float32),
            pltpu.VMEM((NUM_GRAPHS, 128), jnp.float32),
        ],
    )(h, agg, invcnt, root, W, b, onehotT, lin_w, lin_b)


def _aggregate(h, src, dst, edge_type):
    gathered = h[src]
    aggs = []
    for r in range(NUM_REL):
        mask = (edge_type == r).astype(jnp.float32)
        aggs.append(jax.ops.segment_sum(gathered * mask[:, None], dst,
                                        num_segments=NP))
    return jnp.stack(aggs, axis=1)


def _counts(src, dst, edge_type):
    cnts = []
    for r in range(NUM_REL):
        mask = (edge_type == r).astype(jnp.float32)
        cnts.append(jax.ops.segment_sum(mask, dst, num_segments=NP))
    cnt = jnp.stack(cnts, axis=1)
    return 1.0 / jnp.maximum(cnt, 1.0)


def kernel(x, edge_index, edge_attr, edge_type, batch, W1, root1, b1,
           W2, root2, b2, W3, root3, b3, lin_w, lin_b):
    src, dst = edge_index[0], edge_index[1]
    xp = jnp.zeros((NP, x.shape[1]), jnp.float32).at[:N_NODES].set(x)
    batch_pad = jnp.full((NP,), NUM_GRAPHS, jnp.int32).at[:N_NODES].set(batch)
    onehotT = (batch_pad[:, None]
               == jnp.arange(NUM_GRAPHS)[None, :]).astype(jnp.float32)

    invcnt = _counts(src, dst, edge_type)

    agg1 = _aggregate(xp, src, dst, edge_type)
    h1 = _tc_mid_layer(xp, agg1, invcnt, root1, W1, b1.reshape(1, HIDDEN))
    agg2 = _aggregate(h1, src, dst, edge_type)
    h2 = _tc_mid_layer(h1, agg2, invcnt, root2, W2, b2.reshape(1, HIDDEN))
    agg3 = _aggregate(h2, src, dst, edge_type)
    return _tc_last_layer(h2, agg3, invcnt, root3, W3, b3.reshape(1, HIDDEN),
                          onehotT, lin_w, lin_b.reshape(1, -1))


# speedup vs baseline: 1.0069x; 1.0069x over previous
import functools

import jax
import jax.numpy as jnp
from jax import lax
from jax.experimental import pallas as pl
from jax.experimental.pallas import tpu as pltpu
from jax.experimental.pallas import tpu_sc as plsc

N_NODES = 10000
N_EDGES = 160000
NUM_REL = 2
NUM_GRAPHS = 64
HIDDEN = 512
NP = 10240
RB = 320
NSUB = 32
EPAD = N_EDGES + 256
C1 = 2000
OBUF = 3328
K2 = 128
ACC_ROWS = 648
TRASH = 640 * 128


def _wid():
    return lax.axis_index("s") * 2 + lax.axis_index("c")


def _sc_bucket(src, dst, etype):
    mesh = plsc.VectorSubcoreMesh(core_axis_name="c", subcore_axis_name="s")

    @functools.partial(
        pl.kernel,
        out_type=(
            jax.ShapeDtypeStruct((NSUB, EPAD), jnp.int32),
            jax.ShapeDtypeStruct((NSUB, EPAD), jnp.int32),
            jax.ShapeDtypeStruct((NSUB, 16), jnp.int32),
        ),
        mesh=mesh,
        scratch_types=[
            pltpu.VMEM((C1,), jnp.int32),
            pltpu.VMEM((C1,), jnp.int32),
            pltpu.VMEM((C1,), jnp.int32),
            pltpu.VMEM((OBUF,), jnp.int32),
            pltpu.VMEM((OBUF,), jnp.int32),
            pltpu.VMEM((16,), jnp.int32),
        ],
    )
    def k(src_hbm, dst_hbm, typ_hbm, srcc_hbm, slotc_hbm, cnt_hbm,
          sbuf, dbuf, tbuf, osrc, oslot, cvec):
        w = _wid()
        base = w * RB

        def chunk_body(i, carry):
            ocnt, flushed = carry
            pltpu.sync_copy(src_hbm.at[pl.ds(i * C1, C1)], sbuf)
            pltpu.sync_copy(dst_hbm.at[pl.ds(i * C1, C1)], dbuf)
            pltpu.sync_copy(typ_hbm.at[pl.ds(i * C1, C1)], tbuf)

            def grp(g, ocnt):
                off = g * 16
                d = dbuf[pl.ds(off, 16)]
                s = sbuf[pl.ds(off, 16)]
                t = tbuf[pl.ds(off, 16)]
                m = (d >= base) & (d < base + RB)
                slot = (d - base) * 256 + t * 128
                plsc.store_compressed(osrc.at[pl.ds(ocnt, 16)], s, mask=m)
                plsc.store_compressed(oslot.at[pl.ds(ocnt, 16)], slot, mask=m)
                return ocnt + jnp.max(plsc.all_reduce_population_count(m))

            ocnt = lax.fori_loop(0, C1 // 16, grp, ocnt)

            do_flush = ocnt >= 1024

            @pl.when(do_flush)
            def _():
                pltpu.sync_copy(osrc.at[pl.ds(0, 1024)],
                                srcc_hbm.at[w, pl.ds(flushed, 1024)])
                pltpu.sync_copy(oslot.at[pl.ds(0, 1024)],
                                slotc_hbm.at[w, pl.ds(flushed, 1024)])

                def mv(j, _):
                    osrc[pl.ds(j * 16, 16)] = osrc[pl.ds(1024 + j * 16, 16)]
                    oslot[pl.ds(j * 16, 16)] = oslot[pl.ds(1024 + j * 16, 16)]
                    return 0

                lax.fori_loop(0, 128, mv, 0)

            ocnt = jnp.where(do_flush, ocnt - 1024, ocnt)
            flushed = jnp.where(do_flush, flushed + 1024, flushed)
            return ocnt, flushed

        ocnt, flushed = lax.fori_loop(0, N_EDGES // C1, chunk_body,
                                      (jnp.int32(0), jnp.int32(0)))

        zeros = jnp.zeros((16,), jnp.int32)
        trash = jnp.full((16,), TRASH, jnp.int32)
        for kk in range(17):
            osrc[pl.ds(ocnt + kk * 16, 16)] = zeros
            oslot[pl.ds(ocnt + kk * 16, 16)] = trash
        padded = ((lax.max(ocnt, 1) + 255) // 256) * 256

        def fl(j, _):
            pltpu.sync_copy(osrc.at[pl.ds(j * 128, 128)],
                            srcc_hbm.at[w, pl.ds(flushed + j * 128, 128)])
            pltpu.sync_copy(oslot.at[pl.ds(j * 128, 128)],
                            slotc_hbm.at[w, pl.ds(flushed + j * 128, 128)])
            return 0

        lax.fori_loop(0, padded // 128, fl, 0)
        cvec[...] = jnp.full((16,), flushed + padded, jnp.int32)
        pltpu.sync_copy(cvec, cnt_hbm.at[w])

    return k(src, dst, etype)


def _sc_aggregate(h, srcc, slotc, counts, din, do_count, interpret=False):
    Q = din // 128
    h4 = h.reshape(NP * Q, 128)
    mesh = plsc.VectorSubcoreMesh(core_axis_name="c", subcore_axis_name="s")
    out_type = [jax.ShapeDtypeStruct((Q * NP * NUM_REL * 128,), jnp.float32)]
    scratch = [
        pltpu.VMEM((ACC_ROWS * 128,), jnp.float32),
        pltpu.VMEM((K2, 128), jnp.float32),
        pltpu.VMEM((K2, 128), jnp.float32),
        pltpu.VMEM((K2,), jnp.int32),
        pltpu.VMEM((K2,), jnp.int32),
        pltpu.VMEM((K2,), jnp.int32),
        pltpu.VMEM((K2,), jnp.int32),
        pltpu.VMEM((K2,), jnp.int32),
        pltpu.VMEM((K2,), jnp.int32),
        pltpu.VMEM((16,), jnp.int32),
        pltpu.SemaphoreType.DMA,
        pltpu.SemaphoreType.DMA,
    ]
    if do_count:
        out_type.append(jax.ShapeDtypeStruct((NP * NUM_REL * 16,), jnp.float32))
        scratch.append(pltpu.VMEM((ACC_ROWS * 16,), jnp.float32))

    def body(h4_hbm, srcc_hbm, slotc_hbm, cnt_hbm, agg_hbm, *rest):
        if do_count:
            inv_hbm, rest = rest[0], rest[1:]
        (acc, rows0, rows1, sbuf0, sbuf1, slotb0, slotb1, sidx0, sidx1,
         cntv, sem0, sem1) = rest[:12]
        cacc = rest[12] if do_count else None
        rows = (rows0, rows1)
        sbufs = (sbuf0, sbuf1)
        slotbs = (slotb0, slotb1)
        sidxs = (sidx0, sidx1)
        sems = (sem0, sem1)
        w = _wid()
        base = w * RB
        pltpu.sync_copy(cnt_hbm.at[w], cntv)
        total = jnp.max(cntv[...])
        npairs = total // (2 * K2)

        zf = jnp.zeros((16,), jnp.float32)
        onef = jnp.ones((16,), jnp.float32)

        def load_idx(c, s, q):
            pltpu.sync_copy(srcc_hbm.at[w, pl.ds(c * K2, K2)], sbufs[s])
            pltpu.sync_copy(slotc_hbm.at[w, pl.ds(c * K2, K2)], slotbs[s])

            def ib(j, _):
                v = sbufs[s][pl.ds(j * 16, 16)]
                sidxs[s][pl.ds(j * 16, 16)] = v * Q + q
                return 0

            lax.fori_loop(0, K2 // 16, ib, 0)

        def start_gather(s):
            pltpu.async_copy(h4_hbm.at[sidxs[s]], rows[s], sems[s])

        def wait_gather(s):
            pltpu.make_async_copy(h4_hbm.at[sidxs[s]], rows[s], sems[s]).wait()

        def accumulate(s, count_pass):
            rw = rows[s]
            sl = slotbs[s]

            def eb(j, _):
                for u in range(2):
                    jj = j * 2 + u
                    off = sl[jj]
                    for g in range(8):
                        plsc.addupdate(acc.at[pl.ds(off + g * 16, 16)],
                                       rw[jj, pl.ds(g * 16, 16)])
                    if count_pass:
                        plsc.addupdate(cacc.at[pl.ds(off // 8, 16)], onef)
                return 0

            lax.fori_loop(0, K2 // 2, eb, 0)

        for q in range(Q):
            def zb(i, _):
                for g in range(32):
                    acc[pl.ds(i * 512 + g * 16, 16)] = zf
                return 0

            lax.fori_loop(0, ACC_ROWS * 128 // 512, zb, 0)
            count_pass = do_count and q == 0
            if count_pass:
                def zc(i, _):
                    acc_off = i * 16
                    cacc[pl.ds(acc_off, 16)] = zf
                    return 0

                lax.fori_loop(0, ACC_ROWS, zc, 0)

            load_idx(0, 0, q)
            start_gather(0)
            load_idx(1, 1, q)
            start_gather(1)

            def pb(p, _):
                wait_gather(0)
                accumulate(0, count_pass)

                @pl.when(p < npairs - 1)
                def _():
                    load_idx(2 * p + 2, 0, q)
                    start_gather(0)

                wait_gather(1)
                accumulate(1, count_pass)

                @pl.when(p < npairs - 1)
                def _():
                    load_idx(2 * p + 3, 1, q)
                    start_gather(1)

                return 0

            lax.fori_loop(0, npairs, pb, 0)
            pltpu.sync_copy(
                acc.at[pl.ds(0, RB * NUM_REL * 128)],
                agg_hbm.at[pl.ds(q * NP * NUM_REL * 128
                                 + base * NUM_REL * 128,
                                 RB * NUM_REL * 128)])

        if do_count:
            def iv(i, _):
                v = cacc[pl.ds(i * 16, 16)]
                cacc[pl.ds(i * 16, 16)] = 1.0 / jnp.maximum(v, 1.0)
                return 0

            lax.fori_loop(0, RB * NUM_REL, iv, 0)
            pltpu.sync_copy(cacc.at[pl.ds(0, RB * NUM_REL * 16)],
                            inv_hbm.at[pl.ds(base * NUM_REL * 16,
                                             RB * NUM_REL * 16)])

    f = pl.kernel(body, out_type=tuple(out_type), mesh=mesh,
                  scratch_types=scratch, interpret=interpret)
    res = f(h4, srcc, slotc, counts)
    agg = res[0].reshape(Q, NP, NUM_REL, 128)
    if do_count:
        inv = res[1].reshape(NP, NUM_REL, 16)[:, :, 0]
        return agg, inv
    return agg, None


def _elu(x):
    return jnp.where(x > 0, x, jnp.exp(jnp.minimum(x, 0.0)) - 1.0)


def _rgcn_block(h_ref, agg_ref, inv_ref, root_ref, W_ref, b_ref):
    nq = agg_ref.shape[0]
    out = jnp.dot(h_ref[...], root_ref[...], preferred_element_type=jnp.float32)
    for r in range(NUM_REL):
        icr = inv_ref[:, r][:, None]
        for q in range(nq):
            a = agg_ref[q, :, r, :] * icr
            out = out + jnp.dot(a, W_ref[r, q * 128:(q + 1) * 128, :],
                                preferred_element_type=jnp.float32)
    return _elu(out + b_ref[...])


def _mid_layer_body(h_ref, agg_ref, inv_ref, root_ref, W_ref, b_ref, out_ref):
    out_ref[...] = _rgcn_block(h_ref, agg_ref, inv_ref, root_ref, W_ref, b_ref)


def _tc_mid_layer(h, agg, invcnt, root, W, b):
    din = h.shape[1]
    nq = din // 128
    return pl.pallas_call(
        _mid_layer_body,
        grid=(NP // RB,),
        in_specs=[
            pl.BlockSpec((RB, din), lambda i: (i, 0)),
            pl.BlockSpec((nq, RB, NUM_REL, 128), lambda i: (0, i, 0, 0)),
            pl.BlockSpec((RB, NUM_REL), lambda i: (i, 0)),
            pl.BlockSpec((din, HIDDEN), lambda i: (0, 0)),
            pl.BlockSpec((NUM_REL, din, HIDDEN), lambda i: (0, 0, 0)),
            pl.BlockSpec((1, HIDDEN), lambda i: (0, 0)),
        ],
        out_specs=pl.BlockSpec((RB, HIDDEN), lambda i: (i, 0)),
        out_shape=jax.ShapeDtypeStruct((NP, HIDDEN), jnp.float32),
    )(h, agg, invcnt, root, W, b)


def _last_layer_body(h_ref, agg_ref, inv_ref, root_ref, W_ref, b_ref,
                     oh_ref, lw_ref, lb_ref, out_ref, sum_scr, cnt_scr):
    i = pl.program_id(0)

    @pl.when(i == 0)
    def _():
        sum_scr[...] = jnp.zeros_like(sum_scr)
        cnt_scr[...] = jnp.zeros_like(cnt_scr)

    hout = _rgcn_block(h_ref, agg_ref, inv_ref, root_ref, W_ref, b_ref)

    oh = oh_ref[...]
    dn = (((0,), (0,)), ((), ()))
    sum_scr[...] += lax.dot_general(oh, hout, dn,
                                    preferred_element_type=jnp.float32)
    cnt_scr[...] += lax.dot_general(
        oh, jnp.ones((RB, 128), jnp.float32), dn,
        preferred_element_type=jnp.float32)

    @pl.when(i == pl.num_programs(0) - 1)
    def _():
        pooled = sum_scr[...] / jnp.maximum(cnt_scr[:, :1], 1.0)
        out_ref[...] = (jnp.dot(pooled, lw_ref[...],
                                preferred_element_type=jnp.float32)
                        + lb_ref[...])


def _tc_last_layer(h, agg, invcnt, root, W, b, onehotT, lin_w, lin_b):
    din = h.shape[1]
    ncls = lin_w.shape[1]
    return pl.pallas_call(
        _last_layer_body,
        grid=(NP // RB,),
        in_specs=[
            pl.BlockSpec((RB, din), lambda i: (i, 0)),
            pl.BlockSpec((din // 128, RB, NUM_REL, 128), lambda i: (0, i, 0, 0)),
            pl.BlockSpec((RB, NUM_REL), lambda i: (i, 0)),
            pl.BlockSpec((din, HIDDEN), lambda i: (0, 0)),
            pl.BlockSpec((NUM_REL, din, HIDDEN), lambda i: (0, 0, 0)),
            pl.BlockSpec((1, HIDDEN), lambda i: (0, 0)),
            pl.BlockSpec((RB, NUM_GRAPHS), lambda i: (i, 0)),
            pl.BlockSpec((HIDDEN, ncls), lambda i: (0, 0)),
            pl.BlockSpec((1, ncls), lambda i: (0, 0)),
        ],
        out_specs=pl.BlockSpec((NUM_GRAPHS, ncls), lambda i: (0, 0)),
        out_shape=jax.ShapeDtypeStruct((NUM_GRAPHS, ncls), jnp.float32),
        scratch_shapes=[
            pltpu.VMEM((NUM_GRAPHS, HIDDEN), jnp.float32),
            pltpu.VMEM((NUM_GRAPHS, 128), jnp.float32),
        ],
    )(h, agg, invcnt, root, W, b, onehotT, lin_w, lin_b)


def _aggregate(h, src, dst, edge_type):
    gathered = h[src]
    aggs = []
    for r in range(NUM_REL):
        mask = (edge_type == r).astype(jnp.float32)
        aggs.append(jax.ops.segment_sum(gathered * mask[:, None], dst,
                                        num_segments=NP))
    agg = jnp.stack(aggs, axis=1)
    q = h.shape[1] // 128
    return agg.reshape(NP, NUM_REL, q, 128).transpose(2, 0, 1, 3)


def _counts(src, dst, edge_type):
    cnts = []
    for r in range(NUM_REL):
        mask = (edge_type == r).astype(jnp.float32)
        cnts.append(jax.ops.segment_sum(mask, dst, num_segments=NP))
    cnt = jnp.stack(cnts, axis=1)
    return 1.0 / jnp.maximum(cnt, 1.0)


def kernel(x, edge_index, edge_attr, edge_type, batch, W1, root1, b1,
           W2, root2, b2, W3, root3, b3, lin_w, lin_b):
    src, dst = edge_index[0], edge_index[1]
    xp = jnp.zeros((NP, x.shape[1]), jnp.float32).at[:N_NODES].set(x)
    batch_pad = jnp.full((NP,), NUM_GRAPHS, jnp.int32).at[:N_NODES].set(batch)
    onehotT = (batch_pad[:, None]
               == jnp.arange(NUM_GRAPHS)[None, :]).astype(jnp.float32)

    invcnt = _counts(src, dst, edge_type)

    agg1 = _aggregate(xp, src, dst, edge_type)
    h1 = _tc_mid_layer(xp, agg1, invcnt, root1, W1, b1.reshape(1, HIDDEN))
    agg2 = _aggregate(h1, src, dst, edge_type)
    h2 = _tc_mid_layer(h1, agg2, invcnt, root2, W2, b2.reshape(1, HIDDEN))
    agg3 = _aggregate(h2, src, dst, edge_type)
    return _tc_last_layer(h2, agg3, invcnt, root3, W3, b3.reshape(1, HIDDEN),
                          onehotT, lin_w, lin_b.reshape(1, -1))


# speedup vs baseline: 2.5351x; 2.5178x over previous
import functools

import jax
import jax.numpy as jnp
from jax import lax
from jax.experimental import pallas as pl
from jax.experimental.pallas import tpu as pltpu
from jax.experimental.pallas import tpu_sc as plsc

N_NODES = 10000
N_EDGES = 160000
NUM_REL = 2
NUM_GRAPHS = 64
HIDDEN = 512
NP = 10240
RB = 320
NSUB = 32
EPAD = N_EDGES + 256
C1 = 2000
OBUF = 3328
K2 = 128
ACC_ROWS = 648
TRASH = 640 * 128


def _wid():
    return lax.axis_index("s") * 2 + lax.axis_index("c")


def _sc_bucket(src, dst, etype):
    mesh = plsc.VectorSubcoreMesh(core_axis_name="c", subcore_axis_name="s")

    @functools.partial(
        pl.kernel,
        out_type=(
            jax.ShapeDtypeStruct((NSUB * EPAD,), jnp.int32),
            jax.ShapeDtypeStruct((NSUB * EPAD,), jnp.int32),
            jax.ShapeDtypeStruct((NSUB * 16,), jnp.int32),
        ),
        mesh=mesh,
        compiler_params=pltpu.CompilerParams(needs_layout_passes=False),
        scratch_types=[
            pltpu.VMEM((C1,), jnp.int32),
            pltpu.VMEM((C1,), jnp.int32),
            pltpu.VMEM((C1,), jnp.int32),
            pltpu.VMEM((OBUF,), jnp.int32),
            pltpu.VMEM((OBUF,), jnp.int32),
            pltpu.VMEM((16,), jnp.int32),
        ],
    )
    def k(src_hbm, dst_hbm, typ_hbm, srcc_hbm, slotc_hbm, cnt_hbm,
          sbuf, dbuf, tbuf, osrc, oslot, cvec):
        w = _wid()
        base = w * RB

        def chunk_body(i, carry):
            ocnt, flushed = carry
            pltpu.sync_copy(src_hbm.at[pl.ds(i * C1, C1)], sbuf)
            pltpu.sync_copy(dst_hbm.at[pl.ds(i * C1, C1)], dbuf)
            pltpu.sync_copy(typ_hbm.at[pl.ds(i * C1, C1)], tbuf)

            def grp(g, ocnt):
                off = g * 16
                d = dbuf[pl.ds(off, 16)]
                s = sbuf[pl.ds(off, 16)]
                t = tbuf[pl.ds(off, 16)]
                m = (d >= base) & (d < base + RB)
                slot = (d - base) * 256 + t * 128
                cum = plsc.cumsum(jnp.where(m, 1, 0))
                idx = cum + (ocnt - 1)
                plsc.store_scatter(osrc, [idx], s, mask=m)
                plsc.store_scatter(oslot, [idx], slot, mask=m)
                return ocnt + jnp.max(plsc.all_reduce_population_count(m))

            ocnt = lax.fori_loop(0, C1 // 16, grp, ocnt)

            for _rep in range(2):
                do_flush = ocnt >= 1024

                @pl.when(do_flush)
                def _():
                    o = pl.multiple_of(w * EPAD + flushed, 1024)
                    pltpu.sync_copy(osrc.at[pl.ds(0, 1024)],
                                    srcc_hbm.at[pl.ds(o, 1024)])
                    pltpu.sync_copy(oslot.at[pl.ds(0, 1024)],
                                    slotc_hbm.at[pl.ds(o, 1024)])

                    def mv(j, _):
                        osrc[pl.ds(j * 16, 16)] = osrc[pl.ds(1024 + j * 16, 16)]
                        oslot[pl.ds(j * 16, 16)] = oslot[pl.ds(1024 + j * 16, 16)]
                        return 0

                    lax.fori_loop(0, 128, mv, 0)

                ocnt = jnp.where(do_flush, ocnt - 1024, ocnt)
                flushed = jnp.where(do_flush, flushed + 1024, flushed)
            return ocnt, flushed

        ocnt, flushed = lax.fori_loop(0, N_EDGES // C1, chunk_body,
                                      (jnp.int32(0), jnp.int32(0)))

        zeros = jnp.zeros((16,), jnp.int32)
        trash = jnp.full((16,), TRASH, jnp.int32)
        lane = lax.iota(jnp.int32, 16)
        for kk in range(17):
            idx = lane + (ocnt + kk * 16)
            plsc.store_scatter(osrc, [idx], zeros)
            plsc.store_scatter(oslot, [idx], trash)
        padded = ((lax.max(ocnt, 1) + 255) // 256) * 256

        def fl(j, _):
            o = pl.multiple_of(w * EPAD + flushed + j * 128, 128)
            pltpu.sync_copy(osrc.at[pl.ds(j * 128, 128)],
                            srcc_hbm.at[pl.ds(o, 128)])
            pltpu.sync_copy(oslot.at[pl.ds(j * 128, 128)],
                            slotc_hbm.at[pl.ds(o, 128)])
            return 0

        lax.fori_loop(0, padded // 128, fl, 0)
        cvec[...] = jnp.full((16,), flushed + padded, jnp.int32)
        pltpu.sync_copy(cvec, cnt_hbm.at[pl.ds(pl.multiple_of(w * 16, 16), 16)])

    return k(src, dst, etype)


def _sc_aggregate(h, srcc, slotc, counts, din, do_count, interpret=False):
    Q = din // 128
    h4 = h.reshape(NP * Q, 128)
    mesh = plsc.VectorSubcoreMesh(core_axis_name="c", subcore_axis_name="s")
    out_type = [jax.ShapeDtypeStruct((Q * NP * NUM_REL * 128,), jnp.float32)]
    scratch = [
        pltpu.VMEM((ACC_ROWS * 128,), jnp.float32),
        pltpu.VMEM((K2, 128), jnp.float32),
        pltpu.VMEM((K2, 128), jnp.float32),
        pltpu.VMEM((K2,), jnp.int32),
        pltpu.VMEM((K2,), jnp.int32),
        pltpu.VMEM((K2,), jnp.int32),
        pltpu.VMEM((K2,), jnp.int32),
        pltpu.VMEM((K2,), jnp.int32),
        pltpu.VMEM((K2,), jnp.int32),
        pltpu.VMEM((16,), jnp.int32),
        pltpu.SemaphoreType.DMA,
        pltpu.SemaphoreType.DMA,
    ]
    if do_count:
        out_type.append(jax.ShapeDtypeStruct((NP * NUM_REL * 16,), jnp.float32))
        scratch.append(pltpu.VMEM((ACC_ROWS * 16,), jnp.float32))

    def body(h4_hbm, srcc_hbm, slotc_hbm, cnt_hbm, agg_hbm, *rest):
        if do_count:
            inv_hbm, rest = rest[0], rest[1:]
        (acc, rows0, rows1, sbuf0, sbuf1, slotb0, slotb1, sidx0, sidx1,
         cntv, sem0, sem1) = rest[:12]
        cacc = rest[12] if do_count else None
        rows = (rows0, rows1)
        sbufs = (sbuf0, sbuf1)
        slotbs = (slotb0, slotb1)
        sidxs = (sidx0, sidx1)
        sems = (sem0, sem1)
        w = _wid()
        base = w * RB
        pltpu.sync_copy(cnt_hbm.at[pl.ds(pl.multiple_of(w * 16, 16), 16)], cntv)
        total = jnp.max(cntv[...])
        npairs = total // (2 * K2)

        zf = jnp.zeros((16,), jnp.float32)
        onef = jnp.ones((16,), jnp.float32)

        def load_idx(c, s, q):
            o = pl.multiple_of(w * EPAD + c * K2, K2)
            pltpu.sync_copy(srcc_hbm.at[pl.ds(o, K2)], sbufs[s])
            pltpu.sync_copy(slotc_hbm.at[pl.ds(o, K2)], slotbs[s])

            def ib(j, _):
                v = sbufs[s][pl.ds(j * 16, 16)]
                sidxs[s][pl.ds(j * 16, 16)] = v * Q + q
                return 0

            lax.fori_loop(0, K2 // 16, ib, 0)

        def start_gather(s):
            pltpu.async_copy(h4_hbm.at[sidxs[s]], rows[s], sems[s])

        def wait_gather(s):
            pltpu.make_async_copy(h4_hbm.at[sidxs[s]], rows[s], sems[s]).wait()

        def accumulate(s, count_pass):
            rw = rows[s]
            sl = slotbs[s]

            def eb(j, _):
                sv = sl[pl.ds(j * 16, 16)]
                for u in range(16):
                    off = pl.multiple_of(sv[u], 128)
                    for g in range(8):
                        plsc.addupdate(
                            acc.at[pl.ds(pl.multiple_of(off + g * 16, 16), 16)],
                            rw[j * 16 + u, pl.ds(g * 16, 16)])
                    if count_pass:
                        plsc.addupdate(
                            cacc.at[pl.ds(pl.multiple_of(off // 8, 16), 16)],
                            onef)
                return 0

            lax.fori_loop(0, K2 // 16, eb, 0)

        for q in range(Q):
            def zb(i, _):
                for g in range(32):
                    acc[pl.ds(i * 512 + g * 16, 16)] = zf
                return 0

            lax.fori_loop(0, ACC_ROWS * 128 // 512, zb, 0)
            count_pass = do_count and q == 0
            if count_pass:
                def zc(i, _):
                    acc_off = i * 16
                    cacc[pl.ds(acc_off, 16)] = zf
                    return 0

                lax.fori_loop(0, ACC_ROWS, zc, 0)

            load_idx(0, 0, q)
            start_gather(0)
            load_idx(1, 1, q)
            start_gather(1)

            def pb(p, _):
                wait_gather(0)
                accumulate(0, count_pass)

                @pl.when(p < npairs - 1)
                def _():
                    load_idx(2 * p + 2, 0, q)
                    start_gather(0)

                wait_gather(1)
                accumulate(1, count_pass)

                @pl.when(p < npairs - 1)
                def _():
                    load_idx(2 * p + 3, 1, q)
                    start_gather(1)

                return 0

            lax.fori_loop(0, npairs, pb, 0)
            pltpu.sync_copy(
                acc.at[pl.ds(0, RB * NUM_REL * 128)],
                agg_hbm.at[pl.ds(pl.multiple_of(q * NP * NUM_REL * 128
                                                + base * NUM_REL * 128, 128),
                                 RB * NUM_REL * 128)])

        if do_count:
            def iv(i, _):
                v = cacc[pl.ds(i * 16, 16)]
                cacc[pl.ds(i * 16, 16)] = 1.0 / jnp.maximum(v, 1.0)
                return 0

            lax.fori_loop(0, RB * NUM_REL, iv, 0)
            pltpu.sync_copy(cacc.at[pl.ds(0, RB * NUM_REL * 16)],
                            inv_hbm.at[pl.ds(
                                pl.multiple_of(base * NUM_REL * 16, 16),
                                RB * NUM_REL * 16)])

    f = pl.kernel(body, out_type=tuple(out_type), mesh=mesh,
                  scratch_types=scratch,
                  compiler_params=pltpu.CompilerParams(
                      needs_layout_passes=False),
                  interpret=interpret)
    res = f(h4, srcc, slotc, counts)
    agg = res[0].reshape(Q, NP, NUM_REL, 128)
    if do_count:
        inv = res[1].reshape(NP, NUM_REL, 16)[:, :, 0]
        return agg, inv
    return agg, None


def _elu(x):
    return jnp.where(x > 0, x, jnp.exp(jnp.minimum(x, 0.0)) - 1.0)


def _rgcn_block(h_ref, agg_ref, inv_ref, root_ref, W_ref, b_ref):
    nq = agg_ref.shape[0]
    out = jnp.dot(h_ref[...], root_ref[...], preferred_element_type=jnp.float32)
    for r in range(NUM_REL):
        icr = inv_ref[:, r][:, None]
        for q in range(nq):
            a = agg_ref[q, :, r, :] * icr
            out = out + jnp.dot(a, W_ref[r, q * 128:(q + 1) * 128, :],
                                preferred_element_type=jnp.float32)
    return _elu(out + b_ref[...])


def _mid_layer_body(h_ref, agg_ref, inv_ref, root_ref, W_ref, b_ref, out_ref):
    out_ref[...] = _rgcn_block(h_ref, agg_ref, inv_ref, root_ref, W_ref, b_ref)


def _tc_mid_layer(h, agg, invcnt, root, W, b):
    din = h.shape[1]
    nq = din // 128
    return pl.pallas_call(
        _mid_layer_body,
        grid=(NP // RB,),
        in_specs=[
            pl.BlockSpec((RB, din), lambda i: (i, 0)),
            pl.BlockSpec((nq, RB, NUM_REL, 128), lambda i: (0, i, 0, 0)),
            pl.BlockSpec((RB, NUM_REL), lambda i: (i, 0)),
            pl.BlockSpec((din, HIDDEN), lambda i: (0, 0)),
            pl.BlockSpec((NUM_REL, din, HIDDEN), lambda i: (0, 0, 0)),
            pl.BlockSpec((1, HIDDEN), lambda i: (0, 0)),
        ],
        out_specs=pl.BlockSpec((RB, HIDDEN), lambda i: (i, 0)),
        out_shape=jax.ShapeDtypeStruct((NP, HIDDEN), jnp.float32),
    )(h, agg, invcnt, root, W, b)


def _last_layer_body(h_ref, agg_ref, inv_ref, root_ref, W_ref, b_ref,
                     oh_ref, lw_ref, lb_ref, out_ref, sum_scr, cnt_scr):
    i = pl.program_id(0)

    @pl.when(i == 0)
    def _():
        sum_scr[...] = jnp.zeros_like(sum_scr)
        cnt_scr[...] = jnp.zeros_like(cnt_scr)

    hout = _rgcn_block(h_ref, agg_ref, inv_ref, root_ref, W_ref, b_ref)

    oh = oh_ref[...]
    dn = (((0,), (0,)), ((), ()))
    sum_scr[...] += lax.dot_general(oh, hout, dn,
                                    preferred_element_type=jnp.float32)
    cnt_scr[...] += lax.dot_general(
        oh, jnp.ones((RB, 128), jnp.float32), dn,
        preferred_element_type=jnp.float32)

    @pl.when(i == pl.num_programs(0) - 1)
    def _():
        pooled = sum_scr[...] / jnp.maximum(cnt_scr[:, :1], 1.0)
        out_ref[...] = (jnp.dot(pooled, lw_ref[...],
                                preferred_element_type=jnp.float32)
                        + lb_ref[...])


def _tc_last_layer(h, agg, invcnt, root, W, b, onehotT, lin_w, lin_b):
    din = h.shape[1]
    ncls = lin_w.shape[1]
    return pl.pallas_call(
        _last_layer_body,
        grid=(NP // RB,),
        in_specs=[
            pl.BlockSpec((RB, din), lambda i: (i, 0)),
            pl.BlockSpec((din // 128, RB, NUM_REL, 128), lambda i: (0, i, 0, 0)),
            pl.BlockSpec((RB, NUM_REL), lambda i: (i, 0)),
            pl.BlockSpec((din, HIDDEN), lambda i: (0, 0)),
            pl.BlockSpec((NUM_REL, din, HIDDEN), lambda i: (0, 0, 0)),
            pl.BlockSpec((1, HIDDEN), lambda i: (0, 0)),
            pl.BlockSpec((RB, NUM_GRAPHS), lambda i: (i, 0)),
            pl.BlockSpec((HIDDEN, ncls), lambda i: (0, 0)),
            pl.BlockSpec((1, ncls), lambda i: (0, 0)),
        ],
        out_specs=pl.BlockSpec((NUM_GRAPHS, ncls), lambda i: (0, 0)),
        out_shape=jax.ShapeDtypeStruct((NUM_GRAPHS, ncls), jnp.float32),
        scratch_shapes=[
            pltpu.VMEM((NUM_GRAPHS, HIDDEN), jnp.float32),
            pltpu.VMEM((NUM_GRAPHS, 128), jnp.float32),
        ],
    )(h, agg, invcnt, root, W, b, onehotT, lin_w, lin_b)


def _aggregate(h, src, dst, edge_type):
    gathered = h[src]
    aggs = []
    for r in range(NUM_REL):
        mask = (edge_type == r).astype(jnp.float32)
        aggs.append(jax.ops.segment_sum(gathered * mask[:, None], dst,
                                        num_segments=NP))
    agg = jnp.stack(aggs, axis=1)
    q = h.shape[1] // 128
    return agg.reshape(NP, NUM_REL, q, 128).transpose(2, 0, 1, 3)


def _counts(src, dst, edge_type):
    cnts = []
    for r in range(NUM_REL):
        mask = (edge_type == r).astype(jnp.float32)
        cnts.append(jax.ops.segment_sum(mask, dst, num_segments=NP))
    cnt = jnp.stack(cnts, axis=1)
    return 1.0 / jnp.maximum(cnt, 1.0)


def kernel(x, edge_index, edge_attr, edge_type, batch, W1, root1, b1,
           W2, root2, b2, W3, root3, b3, lin_w, lin_b):
    src, dst = edge_index[0], edge_index[1]
    xp = jnp.zeros((NP, x.shape[1]), jnp.float32).at[:N_NODES].set(x)
    batch_pad = jnp.full((NP,), NUM_GRAPHS, jnp.int32).at[:N_NODES].set(batch)
    onehotT = (batch_pad[:, None]
               == jnp.arange(NUM_GRAPHS)[None, :]).astype(jnp.float32)

    srcc, slotc, counts = _sc_bucket(src, dst, edge_type)

    agg1, invcnt = _sc_aggregate(xp, srcc, slotc, counts, x.shape[1], True)
    h1 = _tc_mid_layer(xp, agg1, invcnt, root1, W1, b1.reshape(1, HIDDEN))
    agg2, _ = _sc_aggregate(h1, srcc, slotc, counts, HIDDEN, False)
    h2 = _tc_mid_layer(h1, agg2, invcnt, root2, W2, b2.reshape(1, HIDDEN))
    agg3, _ = _sc_aggregate(h2, srcc, slotc, counts, HIDDEN, False)
    return _tc_last_layer(h2, agg3, invcnt, root3, W3, b3.reshape(1, HIDDEN),
                          onehotT, lin_w, lin_b.reshape(1, -1))
